# baseline (speedup 1.0000x reference)
import functools

import jax
import jax.numpy as jnp
from jax import lax
from jax.experimental import pallas as pl
from jax.experimental.pallas import tpu as pltpu
from jax.experimental.pallas import tpu_sc as plsc

K = 2048
D = 256
BLK = 2048
LBLK = 2048


def _argmin_block_kernel(z_ref, cb_ref, ids_ref, c2_ref):
    @pl.when(pl.program_id(0) == 0)
    def _prologue():
        cb0 = cb_ref[...]
        c2_ref[...] = jnp.sum(cb0 * cb0, axis=1)[None, :]

    z = z_ref[...]
    cb = cb_ref[...]

    mm = jax.lax.dot_general(
        z, cb, (((1,), (1,)), ((), ())),
        preferred_element_type=jnp.float32,
    )
    z2 = jnp.sum(z * z, axis=1, keepdims=True)
    dist = (z2 - 2.0 * mm) + c2_ref[...]

    minval = jnp.min(dist, axis=1, keepdims=True)
    iota = jax.lax.broadcasted_iota(jnp.int32, (BLK, K), 1)
    ids = jnp.min(jnp.where(dist == minval, iota, K), axis=1)
    ids_ref[...] = ids.reshape(1, 1, BLK)


def _loss_block_kernel(q_ref, z_ref, loss_ref):
    loss_ref[...] = (q_ref[...] - z_ref[...]) ** 2


def _make_sc_gather(n_tok):
    info = plsc.get_sparse_core_info()
    nw = info.num_cores * info.num_subcores
    b_per_w = n_tok // nw
    chunk = min(256, b_per_w)
    n_chunks = b_per_w // chunk
    mesh = plsc.VectorSubcoreMesh(core_axis_name="c", subcore_axis_name="s")

    @functools.partial(
        pl.kernel, mesh=mesh,
        out_type=jax.ShapeDtypeStruct((n_tok, D), jnp.float32),
        scratch_types=[
            pltpu.VMEM((b_per_w,), jnp.int32),
            pltpu.VMEM((chunk, D), jnp.float32),
            pltpu.SemaphoreType.DMA,
        ],
    )
    def gather_kernel(table_hbm, idx_hbm, out_hbm, idx_v, rows_v, sem):
        wid = lax.axis_index("s") * info.num_cores + lax.axis_index("c")
        base = wid * b_per_w
        pltpu.sync_copy(idx_hbm.at[pl.ds(base, b_per_w)], idx_v)
        for c in range(n_chunks):
            pltpu.async_copy(
                table_hbm.at[idx_v.at[pl.ds(c * chunk, chunk)]], rows_v, sem
            ).wait()
            pltpu.sync_copy(rows_v, out_hbm.at[pl.ds(base + c * chunk, chunk)])

    return gather_kernel


@functools.partial(jax.jit, static_argnames=())
def kernel(input, codebook):
    B, T, _ = input.shape
    n_tok = B * T
    n_blk = n_tok // BLK
    z = input.reshape(n_tok, D)

    ids3 = pl.pallas_call(
        _argmin_block_kernel,
        grid=(n_blk,),
        in_specs=[
            pl.BlockSpec((BLK, D), lambda i: (i, 0)),
            pl.BlockSpec((K, D), lambda i: (0, 0)),
        ],
        out_specs=pl.BlockSpec((1, 1, BLK), lambda i: (i, 0, 0)),
        out_shape=jax.ShapeDtypeStruct((n_blk, 1, BLK), jnp.int32),
        scratch_shapes=[
            pltpu.VMEM((1, K), jnp.float32),
        ],
    )(z, codebook)

    ids_flat = ids3.reshape(n_tok)
    q = _make_sc_gather(n_tok)(codebook, ids_flat)

    loss = pl.pallas_call(
        _loss_block_kernel,
        grid=(n_tok // LBLK,),
        in_specs=[
            pl.BlockSpec((LBLK, D), lambda i: (i, 0)),
            pl.BlockSpec((LBLK, D), lambda i: (i, 0)),
        ],
        out_specs=pl.BlockSpec((LBLK, D), lambda i: (i, 0)),
        out_shape=jax.ShapeDtypeStruct((n_tok, D), jnp.float32),
    )(q, z)

    q = q.reshape(B, T, D)
    ids = ids_flat.reshape(B, T)
    loss = loss.reshape(B, T, D)
    return (q, ids, loss, loss)
